# Initial kernel scaffold; baseline (speedup 1.0000x reference)
import jax
import jax.numpy as jnp
from jax.experimental import pallas as pl


def kernel(pred_loc, pred_conf, priors, target_boxes, target_labels):
    raise NotImplementedError("write your pallas kernel here")


# speedup vs baseline: 12.3243x; 12.3243x over previous
import jax
import jax.numpy as jnp
from jax.experimental import pallas as pl
from jax.experimental.pallas import tpu as pltpu

_THRESH = 0.5
_RATIO = 3.0
_V0 = 0.1
_V1 = 0.2
_BISECT_ITERS = 42


def _make_kernel(B, P, C, O, NC, CP):
    def _mbl_kernel(loc_ref, conf_ref, pri_ref, tb_ref, tl_ref,
                    out_l, out_c, out_n, scr_ref):
        b = pl.program_id(0)
        nc = pl.program_id(1)

        @pl.when(jnp.logical_and(b == 0, nc == 0))
        def _init():
            zero = jnp.zeros((1, 1), jnp.float32)
            out_l[...] = zero
            out_c[...] = zero
            out_n[...] = zero

        @pl.when(nc == 0)
        def _match():
            pri = pri_ref[...]
            pcx, pcy, pw, ph = pri[0], pri[1], pri[2], pri[3]
            px1 = pcx - pw / 2.0
            py1 = pcy - ph / 2.0
            px2 = pcx + pw / 2.0
            py2 = pcy + ph / 2.0
            tb = tb_ref[0]
            tx1, ty1, tx2, ty2 = tb[0], tb[1], tb[2], tb[3]
            lblf = tl_ref[0, 0].astype(jnp.float32)

            iw = jnp.maximum(
                jnp.minimum(tx2[:, None], px2[None, :])
                - jnp.maximum(tx1[:, None], px1[None, :]), 0.0)
            ih = jnp.maximum(
                jnp.minimum(ty2[:, None], py2[None, :])
                - jnp.maximum(ty1[:, None], py1[None, :]), 0.0)
            inter = iw * ih
            area_t = ((tx2 - tx1) * (ty2 - ty1))[:, None]
            area_p = ((px2 - px1) * (py2 - py1))[None, :]
            ov = inter / (area_t + area_p - inter)

            jj = jax.lax.broadcasted_iota(jnp.int32, (O, P), 0)
            pp = jax.lax.broadcasted_iota(jnp.int32, (O, P), 1)

            bt_ov = jnp.max(ov, axis=0)
            bt_idx = jnp.min(jnp.where(ov == bt_ov[None, :], jj, O), axis=0)
            mx = jnp.max(ov, axis=1)
            bpi = jnp.min(jnp.where(ov == mx[:, None], pp, P), axis=1)

            eq = bpi[:, None] == pp
            forced = jnp.max(eq.astype(jnp.int32), axis=0)
            fj = jnp.max(jnp.where(eq, jj, -1), axis=0)
            fidx = jnp.where(forced > 0, fj, bt_idx)
            fov = jnp.where(forced > 0, 2.0, bt_ov)

            oh = (fidx[None, :] == jj).astype(jnp.float32)
            mx1 = jnp.sum(oh * tx1[:, None], axis=0)
            my1 = jnp.sum(oh * ty1[:, None], axis=0)
            mx2 = jnp.sum(oh * tx2[:, None], axis=0)
            my2 = jnp.sum(oh * ty2[:, None], axis=0)
            lblg = jnp.sum(oh * lblf[:, None], axis=0)

            confl = jnp.where(fov < _THRESH, 0.0, lblg + 1.0)
            posf = (confl > 0.0).astype(jnp.float32)

            g0 = ((mx1 + mx2) / 2.0 - pcx) / (_V0 * pw)
            g1 = ((my1 + my2) / 2.0 - pcy) / (_V0 * ph)
            g2 = jnp.log((mx2 - mx1) / pw) / _V1
            g3 = jnp.log((my2 - my1) / ph) / _V1

            for c_ in range(NC):
                sl = slice(c_ * CP, (c_ + 1) * CP)
                scr_ref[c_, 0, :] = g0[sl]
                scr_ref[c_, 1, :] = g1[sl]
                scr_ref[c_, 2, :] = g2[sl]
                scr_ref[c_, 3, :] = g3[sl]
                scr_ref[c_, 4, :] = posf[sl]
                scr_ref[c_, 6, :] = confl[sl]

        g = scr_ref[nc]
        posf = g[4]
        lc4 = loc_ref[0, 0]
        ll = jnp.float32(0.0)
        for r in range(4):
            d = lc4[r] - g[r]
            ad = jnp.abs(d)
            sl1 = jnp.where(ad < 1.0, 0.5 * d * d, ad - 0.5)
            ll = ll + jnp.sum(sl1 * posf)

        cf = conf_ref[0]
        m = jnp.max(cf, axis=1)
        s = jnp.sum(jnp.exp(cf - m[:, None]), axis=1)
        lse = jnp.log(s) + m
        lbl = g[6].astype(jnp.int32)
        cc = jax.lax.broadcasted_iota(jnp.int32, (CP, C), 1)
        xl = jnp.sum(jnp.where(cc == lbl[:, None], cf, 0.0), axis=1)
        loss_c = lse - xl
        scr_ref[nc, 5, :] = loss_c * (1.0 - posf)

        out_l[...] = out_l[...] + ll
        out_c[...] = out_c[...] + jnp.sum(loss_c * posf)

        @pl.when(nc == NC - 1)
        def _neg():
            vall = scr_ref[:, 5, :]
            posall = scr_ref[:, 4, :]
            npos = jnp.sum(posall)
            k = jnp.minimum(_RATIO * npos, jnp.float32(P - 1))
            mv = jnp.max(vall)

            def body(_, lh):
                lo, hi = lh
                mid = 0.5 * (lo + hi)
                cnt = jnp.sum((vall > mid).astype(jnp.float32))
                big = cnt > k
                return (jnp.where(big, mid, lo), jnp.where(big, hi, mid))

            _, hi = jax.lax.fori_loop(
                0, _BISECT_ITERS, body,
                (jnp.float32(-1.0), mv.astype(jnp.float32)))
            mask = (vall > hi).astype(jnp.float32)
            cnt_hi = jnp.sum(mask)
            sneg = jnp.sum(vall * mask) + (k - cnt_hi) * hi
            out_c[...] = out_c[...] + sneg
            out_n[...] = out_n[...] + npos

    return _mbl_kernel


def kernel(pred_loc, pred_conf, priors, target_boxes, target_labels):
    B, P, _ = pred_loc.shape
    C = pred_conf.shape[-1]
    O = target_boxes.shape[1]
    NC = 4 if P % 4 == 0 else 1
    CP = P // NC

    loc_t = jnp.transpose(pred_loc.reshape(B, NC, CP, 4), (0, 1, 3, 2))
    tb_t = jnp.transpose(target_boxes, (0, 2, 1))
    tl3 = target_labels.reshape(B, 1, O).astype(jnp.int32)
    pri_t = priors.T

    out_shape = [jax.ShapeDtypeStruct((1, 1), jnp.float32)] * 3
    outs = pl.pallas_call(
        _make_kernel(B, P, C, O, NC, CP),
        grid=(B, NC),
        in_specs=[
            pl.BlockSpec((1, 1, 4, CP), lambda b, c: (b, c, 0, 0)),
            pl.BlockSpec((1, CP, C), lambda b, c: (b, c, 0)),
            pl.BlockSpec((4, P), lambda b, c: (0, 0)),
            pl.BlockSpec((1, 4, O), lambda b, c: (b, 0, 0)),
            pl.BlockSpec((1, 1, O), lambda b, c: (b, 0, 0)),
        ],
        out_specs=[pl.BlockSpec((1, 1), lambda b, c: (0, 0))] * 3,
        out_shape=out_shape,
        scratch_shapes=[pltpu.VMEM((NC, 8, CP), jnp.float32)],
    )(loc_t, pred_conf, pri_t, tb_t, tl3)

    ll, lc, nn = outs
    N = nn[0, 0]
    return (ll[0, 0] / N, lc[0, 0] / N)


# speedup vs baseline: 27.2061x; 2.2075x over previous
import jax
import jax.numpy as jnp
from jax.experimental import pallas as pl
from jax.experimental.pallas import tpu as pltpu

_THRESH = 0.5
_RATIO = 3.0
_V0 = 0.1
_V1 = 0.2
_BISECT_ITERS = 32


def _make_kernel(B, P, C, O):
    def _mbl_kernel(loc_ref, conf_ref, pri_ref, tb_ref, tl_ref,
                    out_l, out_c, out_n):
        b = pl.program_id(0)

        @pl.when(b == 0)
        def _init():
            zero = jnp.zeros((1, 1), jnp.float32)
            out_l[...] = zero
            out_c[...] = zero
            out_n[...] = zero

        pri = pri_ref[...]
        pcx, pcy, pw, ph = pri[0], pri[1], pri[2], pri[3]
        px1 = pcx - pw / 2.0
        py1 = pcy - ph / 2.0
        px2 = pcx + pw / 2.0
        py2 = pcy + ph / 2.0
        tb = tb_ref[0]
        tx1, ty1, tx2, ty2 = tb[0], tb[1], tb[2], tb[3]
        lblf = tl_ref[0, 0].astype(jnp.float32)

        iw = jnp.maximum(
            jnp.minimum(tx2[:, None], px2[None, :])
            - jnp.maximum(tx1[:, None], px1[None, :]), 0.0)
        ih = jnp.maximum(
            jnp.minimum(ty2[:, None], py2[None, :])
            - jnp.maximum(ty1[:, None], py1[None, :]), 0.0)
        inter = iw * ih
        area_t = ((tx2 - tx1) * (ty2 - ty1))[:, None]
        area_p = ((px2 - px1) * (py2 - py1))[None, :]
        ov = inter / (area_t + area_p - inter)

        jj = jax.lax.broadcasted_iota(jnp.int32, (O, P), 0)
        pp = jax.lax.broadcasted_iota(jnp.int32, (O, P), 1)

        bt_ov = jnp.max(ov, axis=0)
        bt_idx = jnp.min(jnp.where(ov == bt_ov[None, :], jj, O), axis=0)
        mx = jnp.max(ov, axis=1)
        bpi = jnp.min(jnp.where(ov == mx[:, None], pp, P), axis=1)

        eq = bpi[:, None] == pp
        forced = jnp.max(eq.astype(jnp.int32), axis=0)
        fj = jnp.max(jnp.where(eq, jj, -1), axis=0)
        fidx = jnp.where(forced > 0, fj, bt_idx)
        fov = jnp.where(forced > 0, 2.0, bt_ov)

        oh = (fidx[None, :] == jj).astype(jnp.float32)
        mx1 = jnp.sum(oh * tx1[:, None], axis=0)
        my1 = jnp.sum(oh * ty1[:, None], axis=0)
        mx2 = jnp.sum(oh * tx2[:, None], axis=0)
        my2 = jnp.sum(oh * ty2[:, None], axis=0)
        lblg = jnp.sum(oh * lblf[:, None], axis=0)

        confl = jnp.where(fov < _THRESH, 0.0, lblg + 1.0)
        posf = (confl > 0.0).astype(jnp.float32)

        g0 = ((mx1 + mx2) / 2.0 - pcx) / (_V0 * pw)
        g1 = ((my1 + my2) / 2.0 - pcy) / (_V0 * ph)
        g2 = jnp.log((mx2 - mx1) / pw) / _V1
        g3 = jnp.log((my2 - my1) / ph) / _V1

        lc4 = loc_ref[0]
        ll = jnp.float32(0.0)
        for r, g in enumerate((g0, g1, g2, g3)):
            d = lc4[r] - g
            ad = jnp.abs(d)
            sl1 = jnp.where(ad < 1.0, 0.5 * d * d, ad - 0.5)
            ll = ll + jnp.sum(sl1 * posf)

        cf = conf_ref[0]
        m = jnp.max(cf, axis=0)
        s = jnp.sum(jnp.exp(cf - m[None, :]), axis=0)
        lse = jnp.log(s) + m
        lbl = confl.astype(jnp.int32)
        ccs = jax.lax.broadcasted_iota(jnp.int32, (C, P), 0)
        xl = jnp.sum(jnp.where(ccs == lbl[None, :], cf, 0.0), axis=0)
        loss_c = lse - xl
        v = loss_c * (1.0 - posf)

        npos = jnp.sum(posf)
        k = jnp.minimum(_RATIO * npos, jnp.float32(P - 1))
        mv = jnp.max(v)

        def body(_, lh):
            lo, hi = lh
            mid = 0.5 * (lo + hi)
            cnt = jnp.sum((v > mid).astype(jnp.float32))
            big = cnt > k
            return (jnp.where(big, mid, lo), jnp.where(big, hi, mid))

        _, hi = jax.lax.fori_loop(
            0, _BISECT_ITERS, body,
            (jnp.float32(-1.0), mv.astype(jnp.float32)))
        mask = (v > hi).astype(jnp.float32)
        cnt_hi = jnp.sum(mask)
        sneg = jnp.sum(v * mask) + (k - cnt_hi) * hi

        out_l[...] = out_l[...] + ll
        out_c[...] = out_c[...] + jnp.sum(loss_c * posf) + sneg
        out_n[...] = out_n[...] + npos

    return _mbl_kernel


def kernel(pred_loc, pred_conf, priors, target_boxes, target_labels):
    B, P, _ = pred_loc.shape
    C = pred_conf.shape[-1]
    O = target_boxes.shape[1]

    loc_t = jnp.transpose(pred_loc, (0, 2, 1))
    conf_t = jnp.transpose(pred_conf, (0, 2, 1))
    tb_t = jnp.transpose(target_boxes, (0, 2, 1))
    tl3 = target_labels.reshape(B, 1, O).astype(jnp.int32)
    pri_t = priors.T

    out_shape = [jax.ShapeDtypeStruct((1, 1), jnp.float32)] * 3
    outs = pl.pallas_call(
        _make_kernel(B, P, C, O),
        grid=(B,),
        in_specs=[
            pl.BlockSpec((1, 4, P), lambda b: (b, 0, 0)),
            pl.BlockSpec((1, C, P), lambda b: (b, 0, 0)),
            pl.BlockSpec((4, P), lambda b: (0, 0)),
            pl.BlockSpec((1, 4, O), lambda b: (b, 0, 0)),
            pl.BlockSpec((1, 1, O), lambda b: (b, 0, 0)),
        ],
        out_specs=[pl.BlockSpec((1, 1), lambda b: (0, 0))] * 3,
        out_shape=out_shape,
    )(loc_t, conf_t, pri_t, tb_t, tl3)

    ll, lc, nn = outs
    N = nn[0, 0]
    return (ll[0, 0] / N, lc[0, 0] / N)


# speedup vs baseline: 30.9899x; 1.1391x over previous
import jax
import jax.numpy as jnp
from jax.experimental import pallas as pl
from jax.experimental.pallas import tpu as pltpu

_THRESH = 0.5
_RATIO = 3.0
_V0 = 0.1
_V1 = 0.2
_BISECT_ITERS = 32


def _make_kernel(B, P, C, O):
    def _mbl_kernel(loc_ref, conf_ref, pri_ref, tb_ref, tl_ref,
                    out_l, out_c, out_n):
        b = pl.program_id(0)

        @pl.when(b == 0)
        def _init():
            zero = jnp.zeros((1, 1), jnp.float32)
            out_l[...] = zero
            out_c[...] = zero
            out_n[...] = zero

        pri = pri_ref[...]
        pcx, pcy, pw, ph = pri[0], pri[1], pri[2], pri[3]
        px1 = pcx - pw / 2.0
        py1 = pcy - ph / 2.0
        px2 = pcx + pw / 2.0
        py2 = pcy + ph / 2.0
        tb = tb_ref[0]
        tx1, ty1, tx2, ty2 = tb[0], tb[1], tb[2], tb[3]
        lblf = tl_ref[0, 0].astype(jnp.float32)

        iw = jnp.maximum(
            jnp.minimum(tx2[:, None], px2[None, :])
            - jnp.maximum(tx1[:, None], px1[None, :]), 0.0)
        ih = jnp.maximum(
            jnp.minimum(ty2[:, None], py2[None, :])
            - jnp.maximum(ty1[:, None], py1[None, :]), 0.0)
        inter = iw * ih
        area_t = ((tx2 - tx1) * (ty2 - ty1))[:, None]
        area_p = ((px2 - px1) * (py2 - py1))[None, :]
        ov = inter / (area_t + area_p - inter)

        jj = jax.lax.broadcasted_iota(jnp.int32, (O, P), 0)
        pp = jax.lax.broadcasted_iota(jnp.int32, (O, P), 1)

        bt_ov = jnp.max(ov, axis=0)
        bt_idx = jnp.min(jnp.where(ov == bt_ov[None, :], jj, O), axis=0)
        mx = jnp.max(ov, axis=1)
        bpi = jnp.min(jnp.where(ov == mx[:, None], pp, P), axis=1)

        eq = bpi[:, None] == pp
        forced = jnp.max(eq.astype(jnp.int32), axis=0)
        fj = jnp.max(jnp.where(eq, jj, -1), axis=0)
        fidx = jnp.where(forced > 0, fj, bt_idx)
        fov = jnp.where(forced > 0, 2.0, bt_ov)

        oh = (fidx[None, :] == jj).astype(jnp.float32)
        coef = jnp.concatenate(
            [tb, lblf[None, :], jnp.zeros((3, O), jnp.float32)], axis=0)
        gath = jax.lax.dot_general(
            coef, oh, (((1,), (0,)), ((), ())),
            preferred_element_type=jnp.float32)
        mx1, my1, mx2, my2, lblg = (gath[0], gath[1], gath[2], gath[3],
                                    gath[4])

        confl = jnp.where(fov < _THRESH, 0.0, lblg + 1.0)
        posf = (confl > 0.0).astype(jnp.float32)

        g0 = ((mx1 + mx2) / 2.0 - pcx) / (_V0 * pw)
        g1 = ((my1 + my2) / 2.0 - pcy) / (_V0 * ph)
        g2 = jnp.log((mx2 - mx1) / pw) / _V1
        g3 = jnp.log((my2 - my1) / ph) / _V1

        lc4 = loc_ref[0]
        ll = jnp.float32(0.0)
        for r, g in enumerate((g0, g1, g2, g3)):
            d = lc4[r] - g
            ad = jnp.abs(d)
            sl1 = jnp.where(ad < 1.0, 0.5 * d * d, ad - 0.5)
            ll = ll + jnp.sum(sl1 * posf)

        cf = conf_ref[0]
        m = jnp.max(cf, axis=0)
        e = jnp.exp(cf - m[None, :])
        lbl = confl.astype(jnp.int32)
        ccs = jax.lax.broadcasted_iota(jnp.int32, (C, P), 0)
        sel = jnp.where(ccs == lbl[None, :], cf, 0.0)
        esel = jnp.concatenate([e, sel], axis=0)
        ri = jax.lax.broadcasted_iota(jnp.int32, (8, 2 * C), 0)
        ci = jax.lax.broadcasted_iota(jnp.int32, (8, 2 * C), 1)
        onesm = jnp.where((ri == 0) == (ci < C), 1.0, 0.0).astype(
            jnp.float32) * jnp.where(ri < 2, 1.0, 0.0)
        red = jax.lax.dot_general(
            onesm, esel, (((1,), (0,)), ((), ())),
            preferred_element_type=jnp.float32)
        s, xl = red[0], red[1]
        lse = jnp.log(s) + m
        loss_c = lse - xl
        v = loss_c * (1.0 - posf)

        npos = jnp.sum(posf)
        k = jnp.minimum(_RATIO * npos, jnp.float32(P - 1))
        mv = jnp.max(v)

        def body(_, lh):
            lo, hi = lh
            mid = 0.5 * (lo + hi)
            cnt = jnp.sum((v > mid).astype(jnp.float32))
            big = cnt > k
            return (jnp.where(big, mid, lo), jnp.where(big, hi, mid))

        _, hi = jax.lax.fori_loop(
            0, _BISECT_ITERS, body,
            (jnp.float32(-1.0), mv.astype(jnp.float32)))
        mask = (v > hi).astype(jnp.float32)
        cnt_hi = jnp.sum(mask)
        sneg = jnp.sum(v * mask) + (k - cnt_hi) * hi

        out_l[...] = out_l[...] + ll
        out_c[...] = out_c[...] + jnp.sum(loss_c * posf) + sneg
        out_n[...] = out_n[...] + npos

    return _mbl_kernel


def kernel(pred_loc, pred_conf, priors, target_boxes, target_labels):
    B, P, _ = pred_loc.shape
    C = pred_conf.shape[-1]
    O = target_boxes.shape[1]

    loc_t = jnp.transpose(pred_loc, (0, 2, 1))
    conf_t = jnp.transpose(pred_conf, (0, 2, 1))
    tb_t = jnp.transpose(target_boxes, (0, 2, 1))
    tl3 = target_labels.reshape(B, 1, O).astype(jnp.int32)
    pri_t = priors.T

    out_shape = [jax.ShapeDtypeStruct((1, 1), jnp.float32)] * 3
    outs = pl.pallas_call(
        _make_kernel(B, P, C, O),
        grid=(B,),
        in_specs=[
            pl.BlockSpec((1, 4, P), lambda b: (b, 0, 0)),
            pl.BlockSpec((1, C, P), lambda b: (b, 0, 0)),
            pl.BlockSpec((4, P), lambda b: (0, 0)),
            pl.BlockSpec((1, 4, O), lambda b: (b, 0, 0)),
            pl.BlockSpec((1, 1, O), lambda b: (b, 0, 0)),
        ],
        out_specs=[pl.BlockSpec((1, 1), lambda b: (0, 0))] * 3,
        out_shape=out_shape,
    )(loc_t, conf_t, pri_t, tb_t, tl3)

    ll, lc, nn = outs
    N = nn[0, 0]
    return (ll[0, 0] / N, lc[0, 0] / N)


# speedup vs baseline: 63.1923x; 2.0391x over previous
import jax
import jax.numpy as jnp
from jax.experimental import pallas as pl
from jax.experimental.pallas import tpu as pltpu

_THRESH = 0.5
_RATIO = 3.0
_V0 = 0.1
_V1 = 0.2
_BISECT_ITERS = 32


def _make_batch_kernel(B, P, C, O):
    def _batch_kernel(loc_ref, conf_ref, pri_ref, tb_ref, tl_ref,
                      part_ref, v_ref):
        pri = pri_ref[...]
        pcx, pcy, pw, ph = pri[0], pri[1], pri[2], pri[3]
        px1 = pcx - pw / 2.0
        py1 = pcy - ph / 2.0
        px2 = pcx + pw / 2.0
        py2 = pcy + ph / 2.0
        tb = tb_ref[0]
        tx1, ty1, tx2, ty2 = tb[0], tb[1], tb[2], tb[3]
        lblf = tl_ref[0, 0].astype(jnp.float32)

        iw = jnp.maximum(
            jnp.minimum(tx2[:, None], px2[None, :])
            - jnp.maximum(tx1[:, None], px1[None, :]), 0.0)
        ih = jnp.maximum(
            jnp.minimum(ty2[:, None], py2[None, :])
            - jnp.maximum(ty1[:, None], py1[None, :]), 0.0)
        inter = iw * ih
        area_t = ((tx2 - tx1) * (ty2 - ty1))[:, None]
        area_p = ((px2 - px1) * (py2 - py1))[None, :]
        ov = inter / (area_t + area_p - inter)

        jj = jax.lax.broadcasted_iota(jnp.int32, (O, P), 0)
        pp = jax.lax.broadcasted_iota(jnp.int32, (O, P), 1)

        bt_ov = jnp.max(ov, axis=0)
        bt_idx = jnp.min(jnp.where(ov == bt_ov[None, :], jj, O), axis=0)
        mx = jnp.max(ov, axis=1)
        bpi = jnp.min(jnp.where(ov == mx[:, None], pp, P), axis=1)

        eq = bpi[:, None] == pp
        forced = jnp.max(eq.astype(jnp.int32), axis=0)
        fj = jnp.max(jnp.where(eq, jj, -1), axis=0)
        fidx = jnp.where(forced > 0, fj, bt_idx)
        fov = jnp.where(forced > 0, 2.0, bt_ov)

        oh = (fidx[None, :] == jj).astype(jnp.float32)
        coef = jnp.concatenate(
            [tb, lblf[None, :], jnp.zeros((3, O), jnp.float32)], axis=0)
        gath = jax.lax.dot_general(
            coef, oh, (((1,), (0,)), ((), ())),
            preferred_element_type=jnp.float32)
        mx1, my1, mx2, my2, lblg = (gath[0], gath[1], gath[2], gath[3],
                                    gath[4])

        confl = jnp.where(fov < _THRESH, 0.0, lblg + 1.0)
        posf = (confl > 0.0).astype(jnp.float32)

        g0 = ((mx1 + mx2) / 2.0 - pcx) / (_V0 * pw)
        g1 = ((my1 + my2) / 2.0 - pcy) / (_V0 * ph)
        g2 = jnp.log((mx2 - mx1) / pw) / _V1
        g3 = jnp.log((my2 - my1) / ph) / _V1

        lc4 = loc_ref[0]
        ll = jnp.float32(0.0)
        for r, g in enumerate((g0, g1, g2, g3)):
            d = lc4[r] - g
            ad = jnp.abs(d)
            sl1 = jnp.where(ad < 1.0, 0.5 * d * d, ad - 0.5)
            ll = ll + jnp.sum(sl1 * posf)

        cf = conf_ref[0]
        m = jnp.max(cf, axis=0)
        e = jnp.exp(cf - m[None, :])
        lbl = confl.astype(jnp.int32)
        ccs = jax.lax.broadcasted_iota(jnp.int32, (C, P), 0)
        sel = jnp.where(ccs == lbl[None, :], cf, 0.0)
        esel = jnp.concatenate([e, sel], axis=0)
        ri = jax.lax.broadcasted_iota(jnp.int32, (8, 2 * C), 0)
        ci = jax.lax.broadcasted_iota(jnp.int32, (8, 2 * C), 1)
        onesm = jnp.where((ri == 0) == (ci < C), 1.0, 0.0).astype(
            jnp.float32) * jnp.where(ri < 2, 1.0, 0.0)
        red = jax.lax.dot_general(
            onesm, esel, (((1,), (0,)), ((), ())),
            preferred_element_type=jnp.float32)
        s, xl = red[0], red[1]
        lse = jnp.log(s) + m
        loss_c = lse - xl

        v_ref[0, 0, :] = loss_c * (1.0 - posf)

        lcp = jnp.sum(loss_c * posf)
        npos = jnp.sum(posf)
        pr = jax.lax.broadcasted_iota(jnp.int32, (8, 128), 0)
        out8 = jnp.where(
            pr == 0, ll, jnp.where(pr == 1, lcp,
                                   jnp.where(pr == 2, npos, 0.0)))
        part_ref[0] = out8.astype(jnp.float32)

    return _batch_kernel


def _make_neg_kernel(B, P):
    def _neg_kernel(part_ref, v_ref, out_l, out_c):
        V = v_ref[:, 0, :]
        pt = part_ref[...]
        ll_b = pt[:, 0, 0:1]
        lcp_b = pt[:, 1, 0:1]
        npos_b = pt[:, 2, 0:1]
        k = jnp.minimum(_RATIO * npos_b, jnp.float32(P - 1))
        mv = jnp.max(V, axis=1, keepdims=True)

        def body(_, lh):
            lo, hi = lh
            mid = 0.5 * (lo + hi)
            cnt = jnp.sum((V > mid).astype(jnp.float32), axis=1,
                          keepdims=True)
            big = cnt > k
            return (jnp.where(big, mid, lo), jnp.where(big, hi, mid))

        _, hi = jax.lax.fori_loop(
            0, _BISECT_ITERS, body,
            (jnp.full((B, 1), -1.0, jnp.float32), mv))
        mask = (V > hi).astype(jnp.float32)
        cnt_hi = jnp.sum(mask, axis=1, keepdims=True)
        sneg = jnp.sum(V * mask, axis=1, keepdims=True) + (k - cnt_hi) * hi

        N = jnp.sum(npos_b)
        out_l[...] = jnp.sum(ll_b).reshape(1, 1) / N
        out_c[...] = (jnp.sum(lcp_b) + jnp.sum(sneg)).reshape(1, 1) / N

    return _neg_kernel


def kernel(pred_loc, pred_conf, priors, target_boxes, target_labels):
    B, P, _ = pred_loc.shape
    C = pred_conf.shape[-1]
    O = target_boxes.shape[1]

    loc_t = jnp.transpose(pred_loc, (0, 2, 1))
    conf_t = jnp.transpose(pred_conf, (0, 2, 1))
    tb_t = jnp.transpose(target_boxes, (0, 2, 1))
    tl3 = target_labels.reshape(B, 1, O).astype(jnp.int32)
    pri_t = priors.T

    part, vmat = pl.pallas_call(
        _make_batch_kernel(B, P, C, O),
        grid=(B,),
        in_specs=[
            pl.BlockSpec((1, 4, P), lambda b: (b, 0, 0)),
            pl.BlockSpec((1, C, P), lambda b: (b, 0, 0)),
            pl.BlockSpec((4, P), lambda b: (0, 0)),
            pl.BlockSpec((1, 4, O), lambda b: (b, 0, 0)),
            pl.BlockSpec((1, 1, O), lambda b: (b, 0, 0)),
        ],
        out_specs=[
            pl.BlockSpec((1, 8, 128), lambda b: (b, 0, 0)),
            pl.BlockSpec((1, 1, P), lambda b: (b, 0, 0)),
        ],
        out_shape=[
            jax.ShapeDtypeStruct((B, 8, 128), jnp.float32),
            jax.ShapeDtypeStruct((B, 1, P), jnp.float32),
        ],
        compiler_params=pltpu.CompilerParams(
            dimension_semantics=("parallel",)),
    )(loc_t, conf_t, pri_t, tb_t, tl3)

    out_l, out_c = pl.pallas_call(
        _make_neg_kernel(B, P),
        out_shape=[jax.ShapeDtypeStruct((1, 1), jnp.float32)] * 2,
    )(part, vmat)

    return (out_l[0, 0], out_c[0, 0])
